# Initial kernel scaffold; baseline (speedup 1.0000x reference)
import jax
import jax.numpy as jnp
from jax.experimental import pallas as pl


def kernel(x, T1, b1, R1, C1, T2, b2, R2, C2, fc1_w, fc1_b, fc2_w, fc2_b):
    raise NotImplementedError("write your pallas kernel here")


# speedup vs baseline: 2.7861x; 2.7861x over previous
import functools

import jax
import jax.numpy as jnp
from jax.experimental import pallas as pl
from jax.experimental.pallas import tpu as pltpu


def _fused_kernel(slab_ref, t1_ref, b1_ref, c1_ref, t2_ref, b2_ref, c2_ref,
                  w1_ref, v1_ref, w2_ref, v2_ref, o_ref, *, num_class, bsz):
    f32, bf16 = jnp.float32, jnp.bfloat16

    slab = slab_ref[...]
    hp, s1 = slab.shape[1], slab.shape[2]
    k = t1_ref.shape[0] // s1
    h1 = hp - (k - 1)
    lhs = jnp.concatenate([slab[:, kh:kh + h1, :] for kh in range(k)],
                          axis=2)
    lhs = lhs.reshape(bsz * h1, k * s1).astype(bf16)
    y = jnp.dot(lhs, t1_ref[...], preferred_element_type=f32) + b1_ref[...]
    y = jnp.maximum(y, 0.0)

    n1 = y.shape[1]
    yr = jnp.max(y.reshape(bsz * (h1 // 2), 2, n1), axis=1)
    cc = jnp.dot(yr.astype(bf16), c1_ref[...],
                 preferred_element_type=f32)
    s2 = cc.shape[1] // 2
    p1 = jnp.maximum(cc[:, :s2], cc[:, s2:])
    p1 = p1.reshape(bsz, h1 // 2, s2)

    zpad = jnp.zeros((bsz, 2, s2), f32)
    slab2 = jnp.concatenate([zpad, p1, zpad], axis=1)
    h2 = slab2.shape[1] - (k - 1)
    lhs2 = jnp.concatenate([slab2[:, kh:kh + h2, :] for kh in range(k)],
                           axis=2).reshape(bsz * h2, k * s2).astype(bf16)
    y2 = jnp.dot(lhs2, t2_ref[...], preferred_element_type=f32) + b2_ref[...]
    y2 = jnp.maximum(y2, 0.0)

    n2 = y2.shape[1]
    yr2 = jnp.max(y2.reshape(bsz * (h2 // 2), 2, n2), axis=1)
    cc2 = jnp.dot(yr2.astype(bf16), c2_ref[...],
                  preferred_element_type=f32)
    f2 = cc2.shape[1] // 2
    p2 = jnp.maximum(cc2[:, :f2], cc2[:, f2:])
    p2 = p2.reshape(bsz, h2 // 2, f2).astype(bf16)

    rows = h2 // 2
    acc = jnp.broadcast_to(v1_ref[...], (bsz, v1_ref.shape[1])).astype(f32)
    for yy in range(rows):
        acc = acc + jnp.dot(p2[:, yy, :], w1_ref[yy * f2:(yy + 1) * f2, :],
                            preferred_element_type=f32)
    hid = jnp.maximum(acc, 0.0)

    logits = jnp.dot(hid.astype(bf16), w2_ref[...],
                     preferred_element_type=f32) + v2_ref[...]
    col = jax.lax.broadcasted_iota(jnp.int32, logits.shape, 1)
    logits = jnp.where(col < num_class, logits, -1e30)
    m = jnp.max(logits, axis=-1, keepdims=True)
    sh = logits - m
    lse = jnp.log(jnp.sum(jnp.exp(sh), axis=-1, keepdims=True))
    o_ref[...] = (sh - lse).astype(o_ref.dtype)


def kernel(x, T1, b1, R1, C1, T2, b2, R2, C2, fc1_w, fc1_b, fc2_w, fc2_b):
    del R1, R2
    N, cin, H, W = x.shape
    pad, ksize = 2, 5
    slab1 = T1.shape[0] // ksize
    Hp, Wp = H + 2 * pad, W + 2 * pad
    npad = fc2_w.shape[1]
    num_class = 100

    bsz = 16 if N % 16 == 0 else (8 if N % 8 == 0 else 1)

    xt = jnp.transpose(x, (0, 2, 3, 1))
    xp = jnp.pad(xt, ((0, 0), (pad, pad), (pad, pad), (0, 0)))
    slab = jnp.pad(xp.reshape(N, Hp, Wp * cin),
                   ((0, 0), (0, 0), (0, slab1 - Wp * cin)))

    consts = (T1, b1, C1.astype(jnp.bfloat16), T2, b2,
              C2.astype(jnp.bfloat16), fc1_w, fc1_b, fc2_w, fc2_b)

    def const_spec(a):
        nd = a.ndim
        return pl.BlockSpec(a.shape, lambda n, _nd=nd: (0,) * _nd)

    out = pl.pallas_call(
        functools.partial(_fused_kernel, num_class=num_class, bsz=bsz),
        out_shape=jax.ShapeDtypeStruct((N, npad), jnp.float32),
        grid=(N // bsz,),
        in_specs=[pl.BlockSpec((bsz, Hp, slab1), lambda n: (n, 0, 0))]
                 + [const_spec(a) for a in consts],
        out_specs=pl.BlockSpec((bsz, npad), lambda n: (n, 0)),
        compiler_params=pltpu.CompilerParams(dimension_semantics=("parallel",)),
    )(slab, *consts)
    return out[:, :num_class]


# speedup vs baseline: 2.9020x; 1.0416x over previous
import functools

import jax
import jax.numpy as jnp
from jax.experimental import pallas as pl
from jax.experimental.pallas import tpu as pltpu


def _fused_kernel(q0_ref, q1_ref, q2_ref, q3_ref, t1_ref, b1_ref, c1_ref,
                  t2_ref, b2_ref, c2_ref, w1_ref, v1_ref, w2_ref, v2_ref,
                  o_ref, *, num_class, bsz):
    f32, bf16 = jnp.float32, jnp.bfloat16
    q = (q0_ref[...], q1_ref[...], q2_ref[...], q3_ref[...])
    hq = 16

    def lhs_phase(c):
        parts = []
        for kh in range(5):
            m, o = (c + kh) % 4, (c + kh) // 4
            parts.append(q[m][:, o:o + hq, :])
        return jnp.concatenate(parts, axis=2).reshape(bsz * hq, -1)

    y = [jnp.dot(lhs_phase(c), t1_ref[...], preferred_element_type=f32)
         for c in range(4)]
    p_e = jnp.maximum(jnp.maximum(y[0], y[1]) + b1_ref[...], 0.0)
    p_o = jnp.maximum(jnp.maximum(y[2], y[3]) + b1_ref[...], 0.0)

    cc_e = jnp.dot(p_e.astype(bf16), c1_ref[...], preferred_element_type=f32)
    cc_o = jnp.dot(p_o.astype(bf16), c1_ref[...], preferred_element_type=f32)
    s2 = cc_e.shape[1] // 2
    p1_e = jnp.maximum(cc_e[:, :s2], cc_e[:, s2:]).astype(bf16)
    p1_o = jnp.maximum(cc_o[:, :s2], cc_o[:, s2:]).astype(bf16)
    p1_e = p1_e.reshape(bsz, hq, s2)
    p1_o = p1_o.reshape(bsz, hq, s2)

    z1 = jnp.zeros((bsz, 1, s2), bf16)
    ce = jnp.concatenate([z1, p1_e, z1], axis=1)
    co = jnp.concatenate([z1, p1_o, z1], axis=1)
    lhs2_e = jnp.concatenate(
        [ce[:, 0:16], co[:, 0:16], ce[:, 1:17], co[:, 1:17], ce[:, 2:18]],
        axis=2).reshape(bsz * hq, -1)
    lhs2_o = jnp.concatenate(
        [co[:, 0:16], ce[:, 1:17], co[:, 1:17], ce[:, 2:18], co[:, 2:18]],
        axis=2).reshape(bsz * hq, -1)
    y2_e = jnp.dot(lhs2_e, t2_ref[...], preferred_element_type=f32)
    y2_o = jnp.dot(lhs2_o, t2_ref[...], preferred_element_type=f32)
    p2r = jnp.maximum(jnp.maximum(y2_e, y2_o) + b2_ref[...], 0.0)

    cc2 = jnp.dot(p2r.astype(bf16), c2_ref[...], preferred_element_type=f32)
    f2 = cc2.shape[1] // 2
    p2 = jnp.maximum(cc2[:, :f2], cc2[:, f2:]).astype(bf16)
    p2 = p2.reshape(bsz, hq, f2)

    acc = jnp.broadcast_to(v1_ref[...], (bsz, v1_ref.shape[1])).astype(f32)
    for yy in range(hq):
        acc = acc + jnp.dot(p2[:, yy, :], w1_ref[yy * f2:(yy + 1) * f2, :],
                            preferred_element_type=f32)
    hid = jnp.maximum(acc, 0.0)

    logits = jnp.dot(hid.astype(bf16), w2_ref[...],
                     preferred_element_type=f32) + v2_ref[...]
    col = jax.lax.broadcasted_iota(jnp.int32, logits.shape, 1)
    logits = jnp.where(col < num_class, logits, -1e30)
    m = jnp.max(logits, axis=-1, keepdims=True)
    sh = logits - m
    lse = jnp.log(jnp.sum(jnp.exp(sh), axis=-1, keepdims=True))
    o_ref[...] = (sh - lse).astype(o_ref.dtype)


def kernel(x, T1, b1, R1, C1, T2, b2, R2, C2, fc1_w, fc1_b, fc2_w, fc2_b):
    del R1, R2
    N, cin, H, W = x.shape
    pad, ksize = 2, 5
    slab1 = T1.shape[0] // ksize
    Hp, Wp = H + 2 * pad, W + 2 * pad
    npad = fc2_w.shape[1]
    num_class = 100
    bsz = 16 if N % 16 == 0 else (8 if N % 8 == 0 else 1)

    xt = jnp.transpose(x, (0, 2, 3, 1))
    xp = jnp.pad(xt, ((0, 0), (pad, pad), (pad, pad), (0, 0)))
    slab = jnp.pad(xp.reshape(N, Hp, Wp * cin),
                   ((0, 0), (0, 0), (0, slab1 - Wp * cin))).astype(jnp.bfloat16)
    qs = [slab[:, c::4, :] for c in range(4)]

    consts = (T1, b1, C1.astype(jnp.bfloat16), T2, b2,
              C2.astype(jnp.bfloat16), fc1_w, fc1_b, fc2_w, fc2_b)

    def const_spec(a):
        nd = a.ndim
        return pl.BlockSpec(a.shape, lambda n, _nd=nd: (0,) * _nd)

    hq = (Hp + 3) // 4
    out = pl.pallas_call(
        functools.partial(_fused_kernel, num_class=num_class, bsz=bsz),
        out_shape=jax.ShapeDtypeStruct((N, npad), jnp.float32),
        grid=(N // bsz,),
        in_specs=[pl.BlockSpec((bsz, hq, slab1), lambda n: (n, 0, 0))] * 4
                 + [const_spec(a) for a in consts],
        out_specs=pl.BlockSpec((bsz, npad), lambda n: (n, 0)),
        compiler_params=pltpu.CompilerParams(dimension_semantics=("parallel",)),
    )(*qs, *consts)
    return out[:, :num_class]


# speedup vs baseline: 4.5960x; 1.5837x over previous
import functools

import numpy as np
import jax
import jax.numpy as jnp
from jax.experimental import pallas as pl
from jax.experimental.pallas import tpu as pltpu


def _fused_kernel(q0_ref, q1_ref, q2_ref, q3_ref, p_ref, t1_ref, b1_ref,
                  c1_ref, t2_ref, b2_ref, c2_ref, w1_ref, v1_ref, w2_ref,
                  v2_ref, o_ref, *, num_class, bsz):
    f32, bf16 = jnp.float32, jnp.bfloat16
    hq = 16

    def interleave(qr):
        qq = qr[...]
        cat = jnp.concatenate([qq[:, 0:17], qq[:, 17:34], qq[:, 34:51]],
                              axis=2).reshape(bsz * 17, -1)
        sl = jnp.dot(cat, p_ref[...], preferred_element_type=f32)
        return sl.astype(bf16).reshape(bsz, 17, -1)

    q = (interleave(q0_ref), interleave(q1_ref),
         interleave(q2_ref), interleave(q3_ref))

    def lhs_phase(c):
        parts = []
        for kh in range(5):
            m, o = (c + kh) % 4, (c + kh) // 4
            parts.append(q[m][:, o:o + hq, :])
        return jnp.concatenate(parts, axis=2).reshape(bsz * hq, -1)

    y = [jnp.dot(lhs_phase(c), t1_ref[...], preferred_element_type=f32)
         for c in range(4)]
    p_e = jnp.maximum(jnp.maximum(y[0], y[1]) + b1_ref[...], 0.0)
    p_o = jnp.maximum(jnp.maximum(y[2], y[3]) + b1_ref[...], 0.0)

    cc_e = jnp.dot(p_e.astype(bf16), c1_ref[...], preferred_element_type=f32)
    cc_o = jnp.dot(p_o.astype(bf16), c1_ref[...], preferred_element_type=f32)
    s2 = cc_e.shape[1] // 2
    p1_e = jnp.maximum(cc_e[:, :s2], cc_e[:, s2:]).astype(bf16)
    p1_o = jnp.maximum(cc_o[:, :s2], cc_o[:, s2:]).astype(bf16)
    p1_e = p1_e.reshape(bsz, hq, s2)
    p1_o = p1_o.reshape(bsz, hq, s2)

    z1 = jnp.zeros((bsz, 1, s2), bf16)
    ce = jnp.concatenate([z1, p1_e, z1], axis=1)
    co = jnp.concatenate([z1, p1_o, z1], axis=1)
    lhs2_e = jnp.concatenate(
        [ce[:, 0:16], co[:, 0:16], ce[:, 1:17], co[:, 1:17], ce[:, 2:18]],
        axis=2).reshape(bsz * hq, -1)
    lhs2_o = jnp.concatenate(
        [co[:, 0:16], ce[:, 1:17], co[:, 1:17], ce[:, 2:18], co[:, 2:18]],
        axis=2).reshape(bsz * hq, -1)
    y2_e = jnp.dot(lhs2_e, t2_ref[...], preferred_element_type=f32)
    y2_o = jnp.dot(lhs2_o, t2_ref[...], preferred_element_type=f32)
    p2r = jnp.maximum(jnp.maximum(y2_e, y2_o) + b2_ref[...], 0.0)

    cc2 = jnp.dot(p2r.astype(bf16), c2_ref[...], preferred_element_type=f32)
    f2 = cc2.shape[1] // 2
    p2 = jnp.maximum(cc2[:, :f2], cc2[:, f2:]).astype(bf16)
    p2 = p2.reshape(bsz, hq, f2)

    acc = jnp.broadcast_to(v1_ref[...], (bsz, v1_ref.shape[1])).astype(f32)
    for yy in range(hq):
        acc = acc + jnp.dot(p2[:, yy, :], w1_ref[yy * f2:(yy + 1) * f2, :],
                            preferred_element_type=f32)
    hid = jnp.maximum(acc, 0.0)

    logits = jnp.dot(hid.astype(bf16), w2_ref[...],
                     preferred_element_type=f32) + v2_ref[...]
    col = jax.lax.broadcasted_iota(jnp.int32, logits.shape, 1)
    logits = jnp.where(col < num_class, logits, -1e30)
    m = jnp.max(logits, axis=-1, keepdims=True)
    sh = logits - m
    lse = jnp.log(jnp.sum(jnp.exp(sh), axis=-1, keepdims=True))
    o_ref[...] = (sh - lse).astype(o_ref.dtype)


def kernel(x, T1, b1, R1, C1, T2, b2, R2, C2, fc1_w, fc1_b, fc2_w, fc2_b):
    del R1, R2
    N, cin, H, W = x.shape
    pad, ksize = 2, 5
    slab1 = T1.shape[0] // ksize
    Hp, Wp = H + 2 * pad, W + 2 * pad
    npad = fc2_w.shape[1]
    num_class = 100
    bsz = 16 if N % 16 == 0 else (8 if N % 8 == 0 else 1)

    lanes = 128
    xpl = jnp.pad(x, ((0, 0), (0, 0), (pad, pad),
                      (pad, lanes - W - pad))).astype(jnp.bfloat16)
    qs = [xpl[:, :, m::4, :].reshape(N, cin * ((Hp + 3) // 4), lanes)
          for m in range(4)]

    p_np = np.zeros((cin * lanes, slab1), np.float32)
    for c in range(cin):
        for w in range(Wp):
            p_np[c * lanes + w, w * cin + c] = 1.0
    P = jnp.asarray(p_np, jnp.bfloat16)

    consts = (P, T1, b1, C1.astype(jnp.bfloat16), T2, b2,
              C2.astype(jnp.bfloat16), fc1_w, fc1_b, fc2_w, fc2_b)

    def const_spec(a):
        nd = a.ndim
        return pl.BlockSpec(a.shape, lambda n, _nd=nd: (0,) * _nd)

    hq = (Hp + 3) // 4
    out = pl.pallas_call(
        functools.partial(_fused_kernel, num_class=num_class, bsz=bsz),
        out_shape=jax.ShapeDtypeStruct((N, npad), jnp.float32),
        grid=(N // bsz,),
        in_specs=[pl.BlockSpec((bsz, cin * hq, lanes), lambda n: (n, 0, 0))] * 4
                 + [const_spec(a) for a in consts],
        out_specs=pl.BlockSpec((bsz, npad), lambda n: (n, 0)),
        compiler_params=pltpu.CompilerParams(dimension_semantics=("parallel",)),
    )(*qs, *consts)
    return out[:, :num_class]
